# baseline (speedup 1.0000x reference)
import jax
import jax.numpy as jnp
from jax import lax
from jax.experimental import pallas as pl
from jax.experimental.pallas import tpu as pltpu
from jax.experimental.pallas import tpu_sc as plsc

N_FACTORS = 64
BATCH = 16384
NC = 2
NS = 16
NW = NC * NS
B_PER_W = BATCH // NW
N_CHUNKS = 4
CHUNK = B_PER_W // N_CHUNKS
SUPER = 4
G_PER_SUPER = B_PER_W // SUPER // 16


def _sc_body(uidx_hbm, iidx_hbm, ufacT_hbm, ifacT_hbm, out_hbm,
             uidx_v, iidx_v, ubuf_v, ibuf_v, out_v, sem):
    wid = lax.axis_index("s") * NC + lax.axis_index("c")
    base = wid * B_PER_W

    pltpu.sync_copy(uidx_hbm.at[pl.ds(base, B_PER_W)], uidx_v)
    pltpu.sync_copy(iidx_hbm.at[pl.ds(base, B_PER_W)], iidx_v)

    def gather_row(d, carry):
        for c in range(N_CHUNKS):
            pltpu.async_copy(
                ufacT_hbm.at[d].at[uidx_v.at[pl.ds(c * CHUNK, CHUNK)]],
                ubuf_v.at[d, pl.ds(c * CHUNK, CHUNK)], sem)
            pltpu.async_copy(
                ifacT_hbm.at[d].at[iidx_v.at[pl.ds(c * CHUNK, CHUNK)]],
                ibuf_v.at[d, pl.ds(c * CHUNK, CHUNK)], sem)
        return carry

    lax.fori_loop(0, N_FACTORS, gather_row, 0)

    pltpu.make_async_copy(ufacT_hbm.at[:, pl.ds(0, B_PER_W)], ubuf_v, sem).wait()
    pltpu.make_async_copy(ifacT_hbm.at[:, pl.ds(0, B_PER_W)], ibuf_v, sem).wait()

    def supergroup(s, carry):
        col0 = pl.multiple_of(s * (B_PER_W // SUPER), B_PER_W // SUPER)
        for g in range(G_PER_SUPER):
            acc = jnp.zeros((16,), jnp.float32)
            for d in range(N_FACTORS):
                uv = ubuf_v[d, pl.ds(col0 + g * 16, 16)]
                iv = ibuf_v[d, pl.ds(col0 + g * 16, 16)]
                acc = acc + uv * iv
            out_v[pl.ds(col0 + g * 16, 16)] = acc
        return carry

    lax.fori_loop(0, SUPER, supergroup, 0)

    pltpu.sync_copy(out_v, out_hbm.at[pl.ds(base, B_PER_W)])


@jax.jit
def _run(uidx, iidx, ufacT, ifacT):
    mesh = plsc.VectorSubcoreMesh(core_axis_name="c", subcore_axis_name="s")
    return pl.kernel(
        _sc_body,
        out_type=jax.ShapeDtypeStruct((BATCH,), jnp.float32),
        mesh=mesh,
        compiler_params=pltpu.CompilerParams(use_tc_tiling_on_sc=False),
        scratch_types=[
            pltpu.VMEM((B_PER_W,), jnp.int32),
            pltpu.VMEM((B_PER_W,), jnp.int32),
            pltpu.VMEM((N_FACTORS, B_PER_W), jnp.float32),
            pltpu.VMEM((N_FACTORS, B_PER_W), jnp.float32),
            pltpu.VMEM((B_PER_W,), jnp.float32),
            pltpu.SemaphoreType.DMA,
        ],
    )(uidx, iidx, ufacT, ifacT)


def kernel(user_idx, item_idx, user_factors, item_factors):
    return _run(user_idx.astype(jnp.int32), item_idx.astype(jnp.int32),
                user_factors.T, item_factors.T)


# speedup vs baseline: 3.8091x; 3.8091x over previous
import jax
import jax.numpy as jnp
from jax import lax
from jax.experimental import pallas as pl
from jax.experimental.pallas import tpu as pltpu
from jax.experimental.pallas import tpu_sc as plsc

N_FACTORS = 64
BATCH = 16384
N_ROWS = 1000000
NC = 2
NS = 16
NW = NC * NS
B_PER_W = BATCH // NW
HALF = B_PER_W // 2
N_FULL_BLOCKS = N_ROWS // 128
TAIL = N_ROWS - N_FULL_BLOCKS * 128
BASE_BLOCKS = N_FULL_BLOCKS // NW
EXTRA = N_FULL_BLOCKS - BASE_BLOCKS * NW


def _build_body(ufacT_hbm, ifacT_hbm, tailu_hbm, taili_hbm, comb_hbm,
                bu0, bi0, bu1, bi1, ct, sem0, sem1):
    wid = lax.axis_index("s") * NC + lax.axis_index("c")
    nblk = jnp.where(wid < EXTRA, BASE_BLOCKS + 1, BASE_BLOCKS)
    start = wid * BASE_BLOCKS + jnp.minimum(wid, EXTRA)

    lane = lax.iota(jnp.int32, 16)
    rowvecs = [c * 16 + lane for c in range(8)]

    def fire(blk, bu, bi, sem):
        off = pl.multiple_of(blk * 128, 128)
        pltpu.async_copy(ufacT_hbm.at[:, pl.ds(off, 128)], bu, sem)
        pltpu.async_copy(ifacT_hbm.at[:, pl.ds(off, 128)], bi, sem)

    def wait(bu, bi, sem):
        pltpu.make_async_copy(ufacT_hbm.at[:, pl.ds(0, 128)], bu, sem).wait()
        pltpu.make_async_copy(ifacT_hbm.at[:, pl.ds(0, 128)], bi, sem).wait()

    def process(blk, bu, bi):
        for d in range(N_FACTORS):
            cu = jnp.full((16,), d, jnp.int32)
            ci = jnp.full((16,), d + N_FACTORS, jnp.int32)
            for c in range(8):
                plsc.store_scatter(ct, [rowvecs[c], cu], bu[d, pl.ds(c * 16, 16)])
                plsc.store_scatter(ct, [rowvecs[c], ci], bi[d, pl.ds(c * 16, 16)])
        off = pl.multiple_of(blk * 128, 128)
        pltpu.sync_copy(ct, comb_hbm.at[pl.ds(off, 128), :])

    @pl.when(nblk > 0)
    def _prologue():
        fire(start, bu0, bi0, sem0)

    def pairbody(p, carry):
        b0 = start + p * 2
        b1 = start + p * 2 + 1
        n_done = p * 2

        @pl.when(n_done + 1 < nblk)
        def _f1():
            fire(b1, bu1, bi1, sem1)

        wait(bu0, bi0, sem0)
        process(b0, bu0, bi0)

        @pl.when(n_done + 1 < nblk)
        def _second():
            @pl.when(n_done + 2 < nblk)
            def _f2():
                fire(b1 + 1, bu0, bi0, sem0)

            wait(bu1, bi1, sem1)
            process(b1, bu1, bi1)

        return carry

    lax.fori_loop(0, (nblk + 1) // 2, pairbody, 0)

    @pl.when(wid == NW - 1)
    def _tail():
        off = N_FULL_BLOCKS * 128
        pltpu.sync_copy(tailu_hbm, bu0)
        pltpu.sync_copy(taili_hbm, bi0)
        for d in range(N_FACTORS):
            cu = jnp.full((16,), d, jnp.int32)
            ci = jnp.full((16,), d + N_FACTORS, jnp.int32)
            for c in range(TAIL // 16):
                plsc.store_scatter(ct, [rowvecs[c], cu], bu0[d, pl.ds(c * 16, 16)])
                plsc.store_scatter(ct, [rowvecs[c], ci], bi0[d, pl.ds(c * 16, 16)])
        pltpu.sync_copy(ct.at[pl.ds(0, TAIL), :], comb_hbm.at[pl.ds(off, TAIL), :])


def _dot_body(uidx_hbm, iidx_hbm, comb_hbm, out_hbm,
              uidx_v, iidx_v, uslots, islots, out_v, sem):
    wid = lax.axis_index("s") * NC + lax.axis_index("c")
    base = wid * B_PER_W

    pltpu.sync_copy(uidx_hbm.at[pl.ds(base, B_PER_W)], uidx_v)
    pltpu.sync_copy(iidx_hbm.at[pl.ds(base, B_PER_W)], iidx_v)

    lane = lax.iota(jnp.int32, 16)
    dnums = lax.GatherDimensionNumbers(
        offset_dims=(), collapsed_slice_dims=(0,), start_index_map=(0,))

    def rot(x, k):
        idx = (lane + k) % 16
        return lax.gather(x, idx[:, None], dnums, slice_sizes=(1,),
                          mode=lax.GatherScatterMode.PROMISE_IN_BOUNDS)

    def lane_sum(x):
        for k in (8, 4, 2, 1):
            x = x + rot(x, k)
        return x

    for h in range(2):
        for c in range(HALF // 128):
            pltpu.async_copy(
                comb_hbm.at[uidx_v.at[pl.ds(h * HALF + c * 128, 128)]],
                uslots.at[pl.ds(c * 128, 128), :], sem)
            pltpu.async_copy(
                comb_hbm.at[iidx_v.at[pl.ds(h * HALF + c * 128, 128)]],
                islots.at[pl.ds(c * 128, 128), :], sem)
        pltpu.make_async_copy(comb_hbm.at[pl.ds(0, HALF)], uslots, sem).wait()
        pltpu.make_async_copy(comb_hbm.at[pl.ds(0, HALF)], islots, sem).wait()

        def group(g, carry):
            acc = jnp.zeros((16,), jnp.float32)
            for sub in range(2):
                e0 = pl.multiple_of(g * 16 + sub * 8, 8)
                for r in range(8):
                    p = jnp.zeros((16,), jnp.float32)
                    for k in range(N_FACTORS // 16):
                        uv = uslots[e0 + r, pl.ds(k * 16, 16)]
                        iv = islots[e0 + r, pl.ds(N_FACTORS + k * 16, 16)]
                        p = p + uv * iv
                    acc = jnp.where(lane == sub * 8 + r, lane_sum(p), acc)
            out_v[pl.ds(h * HALF + g * 16, 16)] = acc
            return carry

        lax.fori_loop(0, HALF // 16, group, 0)

    pltpu.sync_copy(out_v, out_hbm.at[pl.ds(base, B_PER_W)])


@jax.jit
def _run(uidx, iidx, ufacT, ifacT, tailu, taili):
    mesh = plsc.VectorSubcoreMesh(core_axis_name="c", subcore_axis_name="s")
    comb = pl.kernel(
        _build_body,
        out_type=jax.ShapeDtypeStruct((N_ROWS, 2 * N_FACTORS), jnp.float32),
        mesh=mesh,
        compiler_params=pltpu.CompilerParams(needs_layout_passes=False),
        scratch_types=[
            pltpu.VMEM((N_FACTORS, 128), jnp.float32),
            pltpu.VMEM((N_FACTORS, 128), jnp.float32),
            pltpu.VMEM((N_FACTORS, 128), jnp.float32),
            pltpu.VMEM((N_FACTORS, 128), jnp.float32),
            pltpu.VMEM((128, 2 * N_FACTORS), jnp.float32),
            pltpu.SemaphoreType.DMA,
            pltpu.SemaphoreType.DMA,
        ],
    )(ufacT, ifacT, tailu, taili)
    return pl.kernel(
        _dot_body,
        out_type=jax.ShapeDtypeStruct((BATCH,), jnp.float32),
        mesh=mesh,
        scratch_types=[
            pltpu.VMEM((B_PER_W,), jnp.int32),
            pltpu.VMEM((B_PER_W,), jnp.int32),
            pltpu.VMEM((HALF, 2 * N_FACTORS), jnp.float32),
            pltpu.VMEM((HALF, 2 * N_FACTORS), jnp.float32),
            pltpu.VMEM((B_PER_W,), jnp.float32),
            pltpu.SemaphoreType.DMA,
        ],
    )(uidx, iidx, comb)


def kernel(user_idx, item_idx, user_factors, item_factors):
    tailu = jnp.pad(user_factors[N_FULL_BLOCKS * 128:], ((0, 128 - TAIL), (0, 0))).T
    taili = jnp.pad(item_factors[N_FULL_BLOCKS * 128:], ((0, 128 - TAIL), (0, 0))).T
    return _run(user_idx.astype(jnp.int32), item_idx.astype(jnp.int32),
                user_factors.T, item_factors.T, tailu, taili)


# speedup vs baseline: 7.9528x; 2.0878x over previous
import jax
import jax.numpy as jnp
from jax import lax
from jax.experimental import pallas as pl
from jax.experimental.pallas import tpu as pltpu
from jax.experimental.pallas import tpu_sc as plsc

N_FACTORS = 64
BATCH = 16384
N_ROWS = 1000000
NC = 2
NS = 16
NW = NC * NS
B_PER_W = BATCH // NW
HALF = B_PER_W // 2
N_FULL_BLOCKS = N_ROWS // 128
TAIL = N_ROWS - N_FULL_BLOCKS * 128
BASE_BLOCKS = N_FULL_BLOCKS // NW
EXTRA = N_FULL_BLOCKS - BASE_BLOCKS * NW


def _build_body(ufacT_hbm, ifacT_hbm, tailu_hbm, taili_hbm, comb_hbm,
                bu0, bi0, bu1, bi1, ct0, ct1, sem0, sem1, wsem0, wsem1):
    wid = lax.axis_index("s") * NC + lax.axis_index("c")
    nblk = jnp.where(wid < EXTRA, BASE_BLOCKS + 1, BASE_BLOCKS)
    start = wid * BASE_BLOCKS + jnp.minimum(wid, EXTRA)

    lane = lax.iota(jnp.int32, 16)
    rowvecs = [c * 16 + lane for c in range(8)]

    def fire(blk, bu, bi, sem):
        off = pl.multiple_of(blk * 128, 128)
        pltpu.async_copy(ufacT_hbm.at[:, pl.ds(off, 128)], bu, sem)
        pltpu.async_copy(ifacT_hbm.at[:, pl.ds(off, 128)], bi, sem)

    def wait(bu, bi, sem):
        pltpu.make_async_copy(ufacT_hbm.at[:, pl.ds(0, 128)], bu, sem).wait()
        pltpu.make_async_copy(ifacT_hbm.at[:, pl.ds(0, 128)], bi, sem).wait()

    def scatter_block(bu, bi, ct, nchunks=8):
        for d in range(N_FACTORS):
            for c in range(nchunks):
                cu = (d + rowvecs[c]) & 127
                ci = (d + N_FACTORS + rowvecs[c]) & 127
                plsc.store_scatter(ct, [rowvecs[c], cu], bu[d, pl.ds(c * 16, 16)])
                plsc.store_scatter(ct, [rowvecs[c], ci], bi[d, pl.ds(c * 16, 16)])

    def write(blk, ct, wsem):
        off = pl.multiple_of(blk * 128, 128)
        pltpu.async_copy(ct, comb_hbm.at[pl.ds(off, 128), :], wsem)

    def wait_write(ct, wsem):
        pltpu.make_async_copy(ct, comb_hbm.at[pl.ds(0, 128), :], wsem).wait()

    @pl.when(nblk > 0)
    def _prologue():
        fire(start, bu0, bi0, sem0)

    def pairbody(p, carry):
        b0 = start + p * 2
        b1 = start + p * 2 + 1
        n_done = p * 2

        @pl.when(n_done + 1 < nblk)
        def _f1():
            fire(b1, bu1, bi1, sem1)

        wait(bu0, bi0, sem0)

        @pl.when(p > 0)
        def _ww0():
            wait_write(ct0, wsem0)

        scatter_block(bu0, bi0, ct0)
        write(b0, ct0, wsem0)

        @pl.when(n_done + 1 < nblk)
        def _second():
            @pl.when(n_done + 2 < nblk)
            def _f2():
                fire(b1 + 1, bu0, bi0, sem0)

            wait(bu1, bi1, sem1)

            @pl.when(p > 0)
            def _ww1():
                wait_write(ct1, wsem1)

            scatter_block(bu1, bi1, ct1)
            write(b1, ct1, wsem1)

        return carry

    lax.fori_loop(0, (nblk + 1) // 2, pairbody, 0)

    wait_write(ct0, wsem0)
    wait_write(ct1, wsem1)

    @pl.when(wid == NW - 1)
    def _tail():
        off = N_FULL_BLOCKS * 128
        pltpu.sync_copy(tailu_hbm, bu0)
        pltpu.sync_copy(taili_hbm, bi0)
        scatter_block(bu0, bi0, ct0, nchunks=TAIL // 16)
        pltpu.sync_copy(ct0.at[pl.ds(0, TAIL), :], comb_hbm.at[pl.ds(off, TAIL), :])


def _dot_body(uidx_hbm, iidx_hbm, comb_hbm, out_hbm,
              uidx_v, iidx_v, uslots, islots, out_v, sem):
    wid = lax.axis_index("s") * NC + lax.axis_index("c")
    base = wid * B_PER_W

    pltpu.sync_copy(uidx_hbm.at[pl.ds(base, B_PER_W)], uidx_v)
    pltpu.sync_copy(iidx_hbm.at[pl.ds(base, B_PER_W)], iidx_v)

    lane = lax.iota(jnp.int32, 16)
    dnums = lax.GatherDimensionNumbers(
        offset_dims=(), collapsed_slice_dims=(0,), start_index_map=(0,))

    def rot(x, k):
        idx = (lane + k) % 16
        return lax.gather(x, idx[:, None], dnums, slice_sizes=(1,),
                          mode=lax.GatherScatterMode.PROMISE_IN_BOUNDS)

    def lane_sum(x):
        for k in (8, 4, 2, 1):
            x = x + rot(x, k)
        return x

    for h in range(2):
        for c in range(HALF // 128):
            pltpu.async_copy(
                comb_hbm.at[uidx_v.at[pl.ds(h * HALF + c * 128, 128)]],
                uslots.at[pl.ds(c * 128, 128), :], sem)
            pltpu.async_copy(
                comb_hbm.at[iidx_v.at[pl.ds(h * HALF + c * 128, 128)]],
                islots.at[pl.ds(c * 128, 128), :], sem)
        pltpu.make_async_copy(comb_hbm.at[pl.ds(0, HALF)], uslots, sem).wait()
        pltpu.make_async_copy(comb_hbm.at[pl.ds(0, HALF)], islots, sem).wait()

        def group(g, carry):
            uvec = uidx_v[pl.ds(h * HALF + g * 16, 16)] & 127
            ivec = iidx_v[pl.ds(h * HALF + g * 16, 16)] & 127
            acc = jnp.zeros((16,), jnp.float32)
            for r in range(16):
                e = g * 16 + r
                ru = uvec[r]
                ri = ivec[r]
                erow = jnp.full((16,), e, jnp.int32)
                p = jnp.zeros((16,), jnp.float32)
                for k in range(N_FACTORS // 16):
                    gu = plsc.load_gather(
                        uslots, [erow, (k * 16 + lane + ru) & 127])
                    gi = plsc.load_gather(
                        islots, [erow, (N_FACTORS + k * 16 + lane + ri) & 127])
                    p = p + gu * gi
                acc = jnp.where(lane == r, lane_sum(p), acc)
            out_v[pl.ds(h * HALF + g * 16, 16)] = acc
            return carry

        lax.fori_loop(0, HALF // 16, group, 0)

    pltpu.sync_copy(out_v, out_hbm.at[pl.ds(base, B_PER_W)])


@jax.jit
def _run(uidx, iidx, ufacT, ifacT, tailu, taili):
    mesh = plsc.VectorSubcoreMesh(core_axis_name="c", subcore_axis_name="s")
    comb = pl.kernel(
        _build_body,
        out_type=jax.ShapeDtypeStruct((N_ROWS, 2 * N_FACTORS), jnp.float32),
        mesh=mesh,
        compiler_params=pltpu.CompilerParams(needs_layout_passes=False),
        scratch_types=[
            pltpu.VMEM((N_FACTORS, 128), jnp.float32),
            pltpu.VMEM((N_FACTORS, 128), jnp.float32),
            pltpu.VMEM((N_FACTORS, 128), jnp.float32),
            pltpu.VMEM((N_FACTORS, 128), jnp.float32),
            pltpu.VMEM((128, 2 * N_FACTORS), jnp.float32),
            pltpu.VMEM((128, 2 * N_FACTORS), jnp.float32),
            pltpu.SemaphoreType.DMA,
            pltpu.SemaphoreType.DMA,
            pltpu.SemaphoreType.DMA,
            pltpu.SemaphoreType.DMA,
        ],
    )(ufacT, ifacT, tailu, taili)
    return pl.kernel(
        _dot_body,
        out_type=jax.ShapeDtypeStruct((BATCH,), jnp.float32),
        mesh=mesh,
        compiler_params=pltpu.CompilerParams(needs_layout_passes=False),
        scratch_types=[
            pltpu.VMEM((B_PER_W,), jnp.int32),
            pltpu.VMEM((B_PER_W,), jnp.int32),
            pltpu.VMEM((HALF, 2 * N_FACTORS), jnp.float32),
            pltpu.VMEM((HALF, 2 * N_FACTORS), jnp.float32),
            pltpu.VMEM((B_PER_W,), jnp.float32),
            pltpu.SemaphoreType.DMA,
        ],
    )(uidx, iidx, comb)


def kernel(user_idx, item_idx, user_factors, item_factors):
    tailu = jnp.pad(user_factors[N_FULL_BLOCKS * 128:], ((0, 128 - TAIL), (0, 0))).T
    taili = jnp.pad(item_factors[N_FULL_BLOCKS * 128:], ((0, 128 - TAIL), (0, 0))).T
    return _run(user_idx.astype(jnp.int32), item_idx.astype(jnp.int32),
                user_factors.T, item_factors.T, tailu, taili)


# speedup vs baseline: 21.6842x; 2.7266x over previous
import functools

import jax
import jax.numpy as jnp
from jax import lax
from jax.experimental import pallas as pl
from jax.experimental.pallas import tpu as pltpu
from jax.experimental.pallas import tpu_sc as plsc

N_FACTORS = 64
BATCH = 16384
N_ROWS = 1000000
NC = 2
NS = 16
NW = NC * NS
B_PER_W = BATCH // NW
HALF = B_PER_W // 2
BLK = 8192
GRID = (N_ROWS + BLK - 1) // BLK


def _transpose_body(uT_ref, iT_ref, out_ref):
    out_ref[:, :N_FACTORS] = uT_ref[...].T
    out_ref[:, N_FACTORS:] = iT_ref[...].T


def _dot_body(uidx_hbm, iidx_hbm, comb_hbm, out_hbm,
              uidx_v, iidx_v, uslots, islots, out_v, sem):
    wid = lax.axis_index("s") * NC + lax.axis_index("c")
    base = wid * B_PER_W

    pltpu.sync_copy(uidx_hbm.at[pl.ds(base, B_PER_W)], uidx_v)
    pltpu.sync_copy(iidx_hbm.at[pl.ds(base, B_PER_W)], iidx_v)

    lane = lax.iota(jnp.int32, 16)
    dnums = lax.GatherDimensionNumbers(
        offset_dims=(), collapsed_slice_dims=(0,), start_index_map=(0,))

    def rot(x, k):
        idx = (lane + k) % 16
        return lax.gather(x, idx[:, None], dnums, slice_sizes=(1,),
                          mode=lax.GatherScatterMode.PROMISE_IN_BOUNDS)

    def lane_sum(x):
        for k in (8, 4, 2, 1):
            x = x + rot(x, k)
        return x

    for h in range(2):
        for c in range(HALF // 128):
            pltpu.async_copy(
                comb_hbm.at[uidx_v.at[pl.ds(h * HALF + c * 128, 128)]],
                uslots.at[pl.ds(c * 128, 128), :], sem)
            pltpu.async_copy(
                comb_hbm.at[iidx_v.at[pl.ds(h * HALF + c * 128, 128)]],
                islots.at[pl.ds(c * 128, 128), :], sem)
        pltpu.make_async_copy(comb_hbm.at[pl.ds(0, HALF)], uslots, sem).wait()
        pltpu.make_async_copy(comb_hbm.at[pl.ds(0, HALF)], islots, sem).wait()

        def group(g, carry):
            acc = jnp.zeros((16,), jnp.float32)
            for sub in range(2):
                e0 = pl.multiple_of(g * 16 + sub * 8, 8)
                for r in range(8):
                    p = jnp.zeros((16,), jnp.float32)
                    for k in range(N_FACTORS // 16):
                        uv = uslots[e0 + r, pl.ds(k * 16, 16)]
                        iv = islots[e0 + r, pl.ds(N_FACTORS + k * 16, 16)]
                        p = p + uv * iv
                    acc = jnp.where(lane == sub * 8 + r, lane_sum(p), acc)
            out_v[pl.ds(h * HALF + g * 16, 16)] = acc
            return carry

        lax.fori_loop(0, HALF // 16, group, 0)

    pltpu.sync_copy(out_v, out_hbm.at[pl.ds(base, B_PER_W)])


@jax.jit
def _run(uidx, iidx, ufacT, ifacT):
    comb = pl.pallas_call(
        _transpose_body,
        grid=(GRID,),
        in_specs=[
            pl.BlockSpec((N_FACTORS, BLK), lambda j: (0, j)),
            pl.BlockSpec((N_FACTORS, BLK), lambda j: (0, j)),
        ],
        out_specs=pl.BlockSpec((BLK, 2 * N_FACTORS), lambda j: (j, 0)),
        out_shape=jax.ShapeDtypeStruct((N_ROWS, 2 * N_FACTORS), jnp.float32),
    )(ufacT, ifacT)

    mesh = plsc.VectorSubcoreMesh(core_axis_name="c", subcore_axis_name="s")
    return pl.kernel(
        _dot_body,
        out_type=jax.ShapeDtypeStruct((BATCH,), jnp.float32),
        mesh=mesh,
        scratch_types=[
            pltpu.VMEM((B_PER_W,), jnp.int32),
            pltpu.VMEM((B_PER_W,), jnp.int32),
            pltpu.VMEM((HALF, 2 * N_FACTORS), jnp.float32),
            pltpu.VMEM((HALF, 2 * N_FACTORS), jnp.float32),
            pltpu.VMEM((B_PER_W,), jnp.float32),
            pltpu.SemaphoreType.DMA,
        ],
    )(uidx, iidx, comb)


def kernel(user_idx, item_idx, user_factors, item_factors):
    return _run(user_idx.astype(jnp.int32), item_idx.astype(jnp.int32),
                user_factors.T, item_factors.T)


# speedup vs baseline: 23.1708x; 1.0686x over previous
import functools

import jax
import jax.numpy as jnp
from jax import lax
from jax.experimental import pallas as pl
from jax.experimental.pallas import tpu as pltpu
from jax.experimental.pallas import tpu_sc as plsc

N_FACTORS = 64
BATCH = 16384
N_ROWS = 1000000
NC = 2
NS = 16
NW = NC * NS
B_PER_W = BATCH // NW
HALF = B_PER_W // 2
BLK = 16384
GRID = (N_ROWS + BLK - 1) // BLK


def _transpose_body(uT_ref, iT_ref, out_ref):
    out_ref[:, :N_FACTORS] = uT_ref[...].T
    out_ref[:, N_FACTORS:] = iT_ref[...].T


def _dot_body(uidx_hbm, iidx_hbm, comb_hbm, out_hbm,
              uidx_v, iidx_v, uslots, islots, out_v, sem):
    wid = lax.axis_index("s") * NC + lax.axis_index("c")
    base = wid * B_PER_W

    pltpu.sync_copy(uidx_hbm.at[pl.ds(base, B_PER_W)], uidx_v)
    pltpu.sync_copy(iidx_hbm.at[pl.ds(base, B_PER_W)], iidx_v)

    lane = lax.iota(jnp.int32, 16)
    dnums = lax.GatherDimensionNumbers(
        offset_dims=(), collapsed_slice_dims=(0,), start_index_map=(0,))

    def rot(x, k):
        idx = (lane + k) % 16
        return lax.gather(x, idx[:, None], dnums, slice_sizes=(1,),
                          mode=lax.GatherScatterMode.PROMISE_IN_BOUNDS)

    def lane_sum(x):
        for k in (8, 4, 2, 1):
            x = x + rot(x, k)
        return x

    for h in range(2):
        for c in range(HALF // 128):
            pltpu.async_copy(
                comb_hbm.at[uidx_v.at[pl.ds(h * HALF + c * 128, 128)]],
                uslots.at[pl.ds(c * 128, 128), :], sem)
            pltpu.async_copy(
                comb_hbm.at[iidx_v.at[pl.ds(h * HALF + c * 128, 128)]],
                islots.at[pl.ds(c * 128, 128), :], sem)
        pltpu.make_async_copy(comb_hbm.at[pl.ds(0, HALF)], uslots, sem).wait()
        pltpu.make_async_copy(comb_hbm.at[pl.ds(0, HALF)], islots, sem).wait()

        def group(g, carry):
            acc = jnp.zeros((16,), jnp.float32)
            for sub in range(2):
                e0 = pl.multiple_of(g * 16 + sub * 8, 8)
                for r in range(8):
                    p = jnp.zeros((16,), jnp.float32)
                    for k in range(N_FACTORS // 16):
                        uv = uslots[e0 + r, pl.ds(k * 16, 16)]
                        iv = islots[e0 + r, pl.ds(N_FACTORS + k * 16, 16)]
                        p = p + uv * iv
                    acc = jnp.where(lane == sub * 8 + r, lane_sum(p), acc)
            out_v[pl.ds(h * HALF + g * 16, 16)] = acc
            return carry

        lax.fori_loop(0, HALF // 16, group, 0)

    pltpu.sync_copy(out_v, out_hbm.at[pl.ds(base, B_PER_W)])


@jax.jit
def _run(uidx, iidx, ufacT, ifacT):
    comb = pl.pallas_call(
        _transpose_body,
        grid=(GRID,),
        in_specs=[
            pl.BlockSpec((N_FACTORS, BLK), lambda j: (0, j)),
            pl.BlockSpec((N_FACTORS, BLK), lambda j: (0, j)),
        ],
        out_specs=pl.BlockSpec((BLK, 2 * N_FACTORS), lambda j: (j, 0)),
        out_shape=jax.ShapeDtypeStruct((N_ROWS, 2 * N_FACTORS), jnp.float32),
    )(ufacT, ifacT)

    mesh = plsc.VectorSubcoreMesh(core_axis_name="c", subcore_axis_name="s")
    return pl.kernel(
        _dot_body,
        out_type=jax.ShapeDtypeStruct((BATCH,), jnp.float32),
        mesh=mesh,
        scratch_types=[
            pltpu.VMEM((B_PER_W,), jnp.int32),
            pltpu.VMEM((B_PER_W,), jnp.int32),
            pltpu.VMEM((HALF, 2 * N_FACTORS), jnp.float32),
            pltpu.VMEM((HALF, 2 * N_FACTORS), jnp.float32),
            pltpu.VMEM((B_PER_W,), jnp.float32),
            pltpu.SemaphoreType.DMA,
        ],
    )(uidx, iidx, comb)


def kernel(user_idx, item_idx, user_factors, item_factors):
    return _run(user_idx.astype(jnp.int32), item_idx.astype(jnp.int32),
                user_factors.T, item_factors.T)
